# baseline (speedup 1.0000x reference)
import jax
import jax.numpy as jnp
from jax import lax
from jax.experimental import pallas as pl
from jax.experimental.pallas import tpu as pltpu
from jax.experimental.pallas import tpu_sc as plsc

D = 16
C = 1024
B = 16384
NC = 2
NS = 16
L = 16
CH = C // NC
RPT = B // NS
CK = 64
NCK = RPT // CK
G = CK // L


def _body(mu_hbm, sig_hbm, mu_t_hbm, sig_t_hbm, idx_hbm,
          out_mu, out_sig,
          idx_v, buf_a, buf_b, accl, cntr, rbuf, ftab, fout,
          part, sem_a, sem_b):
    f32 = jnp.float32
    cid = lax.axis_index("c")
    sid = lax.axis_index("s")
    col0 = cid * CH
    row0 = sid * RPT
    lanes = lax.iota(jnp.int32, L)

    pltpu.sync_copy(idx_hbm.at[pl.ds(row0, RPT)], idx_v)

    def zrow(dd, c):
        for j in range(CH // L):
            accl[dd, pl.ds(j * L, L)] = jnp.zeros((L,), f32)
        return c
    lax.fori_loop(0, 2 * D + 1, zrow, 0)
    cntr[...] = jnp.zeros((L,), f32)

    def compute(buf, i, doff, count):
        def store_batch(pd, pjb, pvals):
            for k in range(16):
                plsc.addupdate(accl.at[pd, pl.ds((pjb + k) * L, L)], pvals[k])

        def grp(g, cm):
            drow = idx_v[pl.ds(i * CK + g * L, L)]
            prev = None
            for rr in range(L):
                d0 = jnp.sum(jnp.where(lanes == rr, drow, 0))
                cm = cm + jnp.where(lanes == d0, 1.0, 0.0)
                d = d0 + doff
                for jb in range(0, CH // L, 16):
                    vals = [buf[g * L + rr, pl.ds((jb + k) * L, L)]
                            for k in range(16)]
                    if prev is not None:
                        store_batch(*prev)
                    prev = (d, jb, vals)
            store_batch(*prev)
            return cm

        cm_fin = plsc.parallel_loop(0, G, carry=jnp.zeros((L,), f32))(grp)
        if count:
            cntr[...] = cntr[...] + cm_fin

    def stream_array(arr_hbm, doff, count):
        def pair(ii, c):
            i0 = 2 * ii
            r0 = row0 + i0 * CK
            pltpu.make_async_copy(
                arr_hbm.at[pl.ds(row0, CK), pl.ds(col0, CH)], buf_a,
                sem_a).wait()
            pltpu.async_copy(
                arr_hbm.at[pl.ds(r0 + CK, CK), pl.ds(col0, CH)], buf_b, sem_b)
            compute(buf_a, i0, doff, count)

            pltpu.make_async_copy(
                arr_hbm.at[pl.ds(row0, CK), pl.ds(col0, CH)], buf_b,
                sem_b).wait()

            @pl.when(ii + 1 < NCK // 2)
            def _():
                pltpu.async_copy(
                    arr_hbm.at[pl.ds(r0 + 2 * CK, CK), pl.ds(col0, CH)],
                    buf_a, sem_a)

            compute(buf_b, i0 + 1, doff, count)
            return c

        pltpu.async_copy(
            arr_hbm.at[pl.ds(row0, CK), pl.ds(col0, CH)], buf_a, sem_a)
        lax.fori_loop(0, NCK // 2, pair, 0)

    stream_array(mu_hbm, 0, True)
    stream_array(sig_hbm, D, False)

    accl[2 * D, pl.ds(0, L)] = cntr[...]
    pltpu.sync_copy(accl, part.at[:, sid])
    plsc.subcore_barrier()

    pltpu.sync_copy(part.at[2 * D], rbuf)
    cntv = rbuf[0, pl.ds(0, L)]
    for t in range(1, NS):
        cntv = cntv + rbuf[t, pl.ds(0, L)]
    cscal = jnp.sum(jnp.where(lanes == sid, cntv, 0.0))
    denomv = jnp.maximum(jnp.full((L,), cscal), 1.0)
    presv = jnp.full((L,), cscal) > 0.0

    pltpu.sync_copy(part.at[sid], rbuf)
    pltpu.sync_copy(mu_t_hbm.at[sid, pl.ds(col0, CH)], ftab)
    for j in range(CH // L):
        s = pl.ds(j * L, L)
        a = rbuf[0, s]
        for t in range(1, NS):
            a = a + rbuf[t, s]
        fout[s] = jnp.where(presv, a / denomv, ftab[s])
    pltpu.sync_copy(fout, out_mu.at[sid, pl.ds(col0, CH)])

    pltpu.sync_copy(part.at[D + sid], rbuf)
    pltpu.sync_copy(sig_t_hbm.at[sid, pl.ds(col0, CH)], ftab)
    for j in range(CH // L):
        s = pl.ds(j * L, L)
        a = rbuf[0, s]
        for t in range(1, NS):
            a = a + rbuf[t, s]
        fout[s] = jnp.where(presv, a / denomv, ftab[s])
    pltpu.sync_copy(fout, out_sig.at[sid, pl.ds(col0, CH)])


@jax.jit
def _run(mu, sig, mu_table, sig_table, domain_idx):
    f32 = jnp.float32
    k = pl.kernel(
        _body,
        out_type=(jax.ShapeDtypeStruct((D, C), f32),
                  jax.ShapeDtypeStruct((D, C), f32)),
        mesh=plsc.VectorSubcoreMesh(core_axis_name="c", subcore_axis_name="s"),
        scratch_types=[
            pltpu.VMEM((RPT,), jnp.int32),
            pltpu.VMEM((CK, CH), f32),
            pltpu.VMEM((CK, CH), f32),
            pltpu.VMEM((2 * D + 1, CH), f32),
            pltpu.VMEM((L,), f32),
            pltpu.VMEM((NS, CH), f32),
            pltpu.VMEM((CH,), f32),
            pltpu.VMEM((CH,), f32),
            pltpu.VMEM_SHARED((2 * D + 1, NS, CH), f32),
            pltpu.SemaphoreType.DMA,
            pltpu.SemaphoreType.DMA,
        ],
        compiler_params=pltpu.CompilerParams(needs_layout_passes=False),
    )
    return k(mu, sig, mu_table, sig_table, domain_idx)


def kernel(mu, sig, mu_table, sig_table, domain_idx, layer_idx):
    del layer_idx
    return _run(mu, sig, mu_table, sig_table, domain_idx)


# speedup vs baseline: 1.0308x; 1.0308x over previous
import jax
import jax.numpy as jnp
from jax import lax
from jax.experimental import pallas as pl
from jax.experimental.pallas import tpu as pltpu
from jax.experimental.pallas import tpu_sc as plsc

D = 16
C = 1024
B = 16384
NC = 2
NS = 16
L = 16
CH = C // NC
RPT = B // NS
CK = 64
NCK = RPT // CK
G = CK // L


def _body(mu_hbm, sig_hbm, mu_t_hbm, sig_t_hbm, idx_hbm,
          out_mu, out_sig,
          idx_v, buf_a, buf_b, accl, cntr, rbuf, ftab, fout,
          part, sem_a, sem_b):
    f32 = jnp.float32
    cid = lax.axis_index("c")
    sid = lax.axis_index("s")
    col0 = cid * CH
    row0 = sid * RPT
    lanes = lax.iota(jnp.int32, L)

    pltpu.async_copy(
        mu_hbm.at[pl.ds(row0, CK), pl.ds(col0, CH)], buf_a, sem_a)
    pltpu.async_copy(
        sig_hbm.at[pl.ds(row0, CK), pl.ds(col0, CH)], buf_b, sem_b)

    pltpu.sync_copy(idx_hbm.at[pl.ds(row0, RPT)], idx_v)

    def zrow(dd, c):
        for j in range(CH // L):
            accl[dd, pl.ds(j * L, L)] = jnp.zeros((L,), f32)
        return c
    lax.fori_loop(0, 2 * D + 1, zrow, 0)
    cntr[...] = jnp.zeros((L,), f32)

    def compute(buf, i, doff, count):
        def store_batch(pd, pjb, pvals):
            for k in range(16):
                plsc.addupdate(accl.at[pd, pl.ds((pjb + k) * L, L)], pvals[k])

        def grp(g, c2):
            drow = idx_v[pl.ds(i * CK + g * L, L)]
            cm = jnp.zeros((L,), f32)
            prev = None
            for rr in range(L):
                d0 = jnp.sum(jnp.where(lanes == rr, drow, 0))
                cm = cm + jnp.where(lanes == d0, 1.0, 0.0)
                d = d0 + doff
                for jb in range(0, CH // L, 16):
                    vals = [buf[g * L + rr, pl.ds((jb + k) * L, L)]
                            for k in range(16)]
                    if prev is not None:
                        store_batch(*prev)
                    prev = (d, jb, vals)
            store_batch(*prev)
            if count:
                cntr[...] = cntr[...] + cm
            return c2

        lax.fori_loop(0, G, grp, 0)

    def chunk(i, c):
        r = row0 + i * CK
        pltpu.make_async_copy(
            mu_hbm.at[pl.ds(row0, CK), pl.ds(col0, CH)], buf_a, sem_a).wait()
        compute(buf_a, i, 0, True)

        @pl.when(i + 1 < NCK)
        def _():
            pltpu.async_copy(
                mu_hbm.at[pl.ds(r + CK, CK), pl.ds(col0, CH)], buf_a, sem_a)

        pltpu.make_async_copy(
            sig_hbm.at[pl.ds(row0, CK), pl.ds(col0, CH)], buf_b, sem_b).wait()
        compute(buf_b, i, D, False)

        @pl.when(i + 1 < NCK)
        def _():
            pltpu.async_copy(
                sig_hbm.at[pl.ds(r + CK, CK), pl.ds(col0, CH)], buf_b, sem_b)

        return c

    lax.fori_loop(0, NCK, chunk, 0)

    accl[2 * D, pl.ds(0, L)] = cntr[...]
    pltpu.sync_copy(accl, part.at[:, sid])
    plsc.subcore_barrier()

    pltpu.sync_copy(part.at[2 * D], rbuf)
    cntv = rbuf[0, pl.ds(0, L)]
    for t in range(1, NS):
        cntv = cntv + rbuf[t, pl.ds(0, L)]
    cscal = jnp.sum(jnp.where(lanes == sid, cntv, 0.0))
    denomv = jnp.maximum(jnp.full((L,), cscal), 1.0)
    presv = jnp.full((L,), cscal) > 0.0

    pltpu.sync_copy(part.at[sid], rbuf)
    pltpu.sync_copy(mu_t_hbm.at[sid, pl.ds(col0, CH)], ftab)
    for j in range(CH // L):
        s = pl.ds(j * L, L)
        a = rbuf[0, s]
        for t in range(1, NS):
            a = a + rbuf[t, s]
        fout[s] = jnp.where(presv, a / denomv, ftab[s])
    pltpu.sync_copy(fout, out_mu.at[sid, pl.ds(col0, CH)])

    pltpu.sync_copy(part.at[D + sid], rbuf)
    pltpu.sync_copy(sig_t_hbm.at[sid, pl.ds(col0, CH)], ftab)
    for j in range(CH // L):
        s = pl.ds(j * L, L)
        a = rbuf[0, s]
        for t in range(1, NS):
            a = a + rbuf[t, s]
        fout[s] = jnp.where(presv, a / denomv, ftab[s])
    pltpu.sync_copy(fout, out_sig.at[sid, pl.ds(col0, CH)])


@jax.jit
def _run(mu, sig, mu_table, sig_table, domain_idx):
    f32 = jnp.float32
    k = pl.kernel(
        _body,
        out_type=(jax.ShapeDtypeStruct((D, C), f32),
                  jax.ShapeDtypeStruct((D, C), f32)),
        mesh=plsc.VectorSubcoreMesh(core_axis_name="c", subcore_axis_name="s"),
        scratch_types=[
            pltpu.VMEM((RPT,), jnp.int32),
            pltpu.VMEM((CK, CH), f32),
            pltpu.VMEM((CK, CH), f32),
            pltpu.VMEM((2 * D + 1, CH), f32),
            pltpu.VMEM((L,), f32),
            pltpu.VMEM((NS, CH), f32),
            pltpu.VMEM((CH,), f32),
            pltpu.VMEM((CH,), f32),
            pltpu.VMEM_SHARED((2 * D + 1, NS, CH), f32),
            pltpu.SemaphoreType.DMA,
            pltpu.SemaphoreType.DMA,
        ],
        compiler_params=pltpu.CompilerParams(needs_layout_passes=False),
    )
    return k(mu, sig, mu_table, sig_table, domain_idx)


def kernel(mu, sig, mu_table, sig_table, domain_idx, layer_idx):
    del layer_idx
    return _run(mu, sig, mu_table, sig_table, domain_idx)
